# baseline (speedup 1.0000x reference)
import functools

import jax
import jax.numpy as jnp
from jax import lax
from jax.experimental import pallas as pl
from jax.experimental.pallas import tpu as pltpu
from jax.experimental.pallas import tpu_sc as plsc

NUM_EMB = 8192
DIM = 256
BM = 256


def _argmin_body(x_ref, e_ref, idx_ref, loss_ref, e2_ref):
    i = pl.program_id(0)

    @pl.when(i == 0)
    def _init():
        e = e_ref[...]
        e2_ref[...] = jnp.sum(e * e, axis=1, keepdims=True).reshape(1, NUM_EMB)
        loss_ref[...] = jnp.zeros((1, 1), jnp.float32)

    x = x_ref[...]
    x2 = jnp.sum(x * x, axis=1, keepdims=True)
    m = lax.dot_general(x, e_ref[...], (((1,), (1,)), ((), ())),
                        preferred_element_type=jnp.float32)
    d = (x2 + e2_ref[...]) - 2.0 * m
    col = lax.broadcasted_iota(jnp.int32, d.shape, 1)
    accv = acci = exact = None
    for lo, hi in ((0, 2736), (2736, 5472), (5472, NUM_EMB)):
        mask = (col >= lo) & (col < hi)
        v = jnp.where(mask, d, jnp.inf)
        mp = jnp.min(v, axis=1, keepdims=True)
        ip = jnp.min(jnp.where(v == mp, col, NUM_EMB), axis=1)
        mp = mp[:, 0]
        mpr = mp.astype(jnp.bfloat16).astype(jnp.float32)
        if accv is None:
            accv, acci, exact = mpr, ip, mp
        else:
            take = mp < accv
            accv = jnp.where(take, mpr, accv)
            acci = jnp.where(take, ip, acci)
            exact = jnp.where(take, mp, exact)
    idx_ref[0, 0, :] = acci
    loss_ref[...] += jnp.sum(exact).reshape(1, 1)


def _vq_argmin_call(flat, embedding):
    n = flat.shape[0]
    nblk = n // BM
    grid = (nblk,)
    idx_out = jax.ShapeDtypeStruct((nblk, 1, BM), jnp.int32)
    loss_out = jax.ShapeDtypeStruct((1, 1), jnp.float32)
    idx3, loss = pl.pallas_call(
        _argmin_body,
        grid=grid,
        in_specs=[
            pl.BlockSpec((BM, DIM), lambda i: (i, 0)),
            pl.BlockSpec((NUM_EMB, DIM), lambda i: (0, 0)),
        ],
        out_specs=[
            pl.BlockSpec((1, 1, BM), lambda i: (i, 0, 0)),
            pl.BlockSpec((1, 1), lambda i: (0, 0)),
        ],
        out_shape=[idx_out, loss_out],
        scratch_shapes=[pltpu.VMEM((1, NUM_EMB), jnp.float32)],
    )(flat, embedding)
    return idx3.reshape(n), loss[0, 0]


def _make_gather(nrows):
    info = plsc.get_sparse_core_info()
    nc, ns = info.num_cores, info.num_subcores
    nw = nc * ns
    b_per_w = nrows // nw
    ch = 128
    n_ch = b_per_w // ch
    mesh = plsc.VectorSubcoreMesh(core_axis_name="c", subcore_axis_name="s")

    @functools.partial(
        pl.kernel,
        mesh=mesh,
        out_type=jax.ShapeDtypeStruct((nrows, DIM), jnp.float32),
        scratch_types=[
            pltpu.VMEM((ch,), jnp.int32),
            pltpu.VMEM((ch, DIM), jnp.float32),
            pltpu.SemaphoreType.DMA,
        ],
    )
    def gather_k(table_hbm, idx_hbm, out_hbm, idx_v, rows_v, sem):
        wid = lax.axis_index("s") * nc + lax.axis_index("c")
        base = wid * b_per_w
        for t in range(n_ch):
            o = base + t * ch
            pltpu.sync_copy(idx_hbm.at[pl.ds(o, ch)], idx_v)
            pltpu.async_copy(table_hbm.at[idx_v], rows_v, sem).wait()
            pltpu.sync_copy(rows_v, out_hbm.at[pl.ds(o, ch)])

    return gather_k


def kernel(inputs, embedding):
    flat = inputs.reshape(-1, DIM)
    n = flat.shape[0]
    idx_flat, loss_sum = _vq_argmin_call(flat, embedding)
    quantized_flat = _make_gather(n)(embedding, idx_flat)
    quantized = quantized_flat.reshape(inputs.shape)
    loss = loss_sum * (1.25 / (n * DIM))
    return (quantized, idx_flat[:, None], loss)


# speedup vs baseline: 1.3733x; 1.3733x over previous
import functools

import jax
import jax.numpy as jnp
from jax import lax
from jax.experimental import pallas as pl
from jax.experimental.pallas import tpu as pltpu
from jax.experimental.pallas import tpu_sc as plsc

NUM_EMB = 8192
DIM = 256
BM = 256


_PARTS = ((0, 2736), (2736, 5472), (5472, NUM_EMB))


def _argmin_body(x_ref, e_ref, idx_ref, loss_ref, e2a_ref, e2b_ref, e2c_ref):
    i = pl.program_id(0)
    e2_refs = (e2a_ref, e2b_ref, e2c_ref)

    @pl.when(i == 0)
    def _init():
        for (lo, hi), r in zip(_PARTS, e2_refs):
            e_p = e_ref[lo:hi, :]
            r[...] = jnp.sum(e_p * e_p, axis=1, keepdims=True).reshape(1, hi - lo)
        loss_ref[...] = jnp.zeros((1, 1), jnp.float32)

    x = x_ref[...]
    x2 = jnp.sum(x * x, axis=1, keepdims=True)
    accv = acci = exact = None
    for (lo, hi), e2_r in zip(_PARTS, e2_refs):
        e_p = e_ref[lo:hi, :]
        m = lax.dot_general(x, e_p, (((1,), (1,)), ((), ())),
                            preferred_element_type=jnp.float32)
        d = (x2 + e2_r[...]) - 2.0 * m
        mp = jnp.min(d, axis=1, keepdims=True)
        col = lax.broadcasted_iota(jnp.int32, d.shape, 1) + lo
        ip = jnp.min(jnp.where(d == mp, col, NUM_EMB), axis=1)
        mp = mp[:, 0]
        mpr = mp.astype(jnp.bfloat16).astype(jnp.float32)
        if accv is None:
            accv, acci, exact = mpr, ip, mp
        else:
            take = mp < accv
            accv = jnp.where(take, mpr, accv)
            acci = jnp.where(take, ip, acci)
            exact = jnp.where(take, mp, exact)
    idx_ref[0, 0, :] = acci
    loss_ref[...] += jnp.sum(exact).reshape(1, 1)


def _vq_argmin_call(flat, embedding):
    n = flat.shape[0]
    nblk = n // BM
    grid = (nblk,)
    idx_out = jax.ShapeDtypeStruct((nblk, 1, BM), jnp.int32)
    loss_out = jax.ShapeDtypeStruct((1, 1), jnp.float32)
    idx3, loss = pl.pallas_call(
        _argmin_body,
        grid=grid,
        in_specs=[
            pl.BlockSpec((BM, DIM), lambda i: (i, 0)),
            pl.BlockSpec((NUM_EMB, DIM), lambda i: (0, 0)),
        ],
        out_specs=[
            pl.BlockSpec((1, 1, BM), lambda i: (i, 0, 0)),
            pl.BlockSpec((1, 1), lambda i: (0, 0)),
        ],
        out_shape=[idx_out, loss_out],
        scratch_shapes=[pltpu.VMEM((1, hi - lo), jnp.float32)
                        for lo, hi in _PARTS],
    )(flat, embedding)
    return idx3.reshape(n), loss[0, 0]


def _make_gather(nrows):
    info = plsc.get_sparse_core_info()
    nc, ns = info.num_cores, info.num_subcores
    nw = nc * ns
    b_per_w = nrows // nw
    ch = 128
    n_ch = b_per_w // ch
    mesh = plsc.VectorSubcoreMesh(core_axis_name="c", subcore_axis_name="s")

    @functools.partial(
        pl.kernel,
        mesh=mesh,
        out_type=jax.ShapeDtypeStruct((nrows, DIM), jnp.float32),
        scratch_types=[
            pltpu.VMEM((ch,), jnp.int32),
            pltpu.VMEM((ch, DIM), jnp.float32),
            pltpu.SemaphoreType.DMA,
        ],
    )
    def gather_k(table_hbm, idx_hbm, out_hbm, idx_v, rows_v, sem):
        wid = lax.axis_index("s") * nc + lax.axis_index("c")
        base = wid * b_per_w
        for t in range(n_ch):
            o = base + t * ch
            pltpu.sync_copy(idx_hbm.at[pl.ds(o, ch)], idx_v)
            pltpu.async_copy(table_hbm.at[idx_v], rows_v, sem).wait()
            pltpu.sync_copy(rows_v, out_hbm.at[pl.ds(o, ch)])

    return gather_k


def kernel(inputs, embedding):
    flat = inputs.reshape(-1, DIM)
    n = flat.shape[0]
    idx_flat, loss_sum = _vq_argmin_call(flat, embedding)
    quantized_flat = _make_gather(n)(embedding, idx_flat)
    quantized = quantized_flat.reshape(inputs.shape)
    loss = loss_sum * (1.25 / (n * DIM))
    return (quantized, idx_flat[:, None], loss)


# speedup vs baseline: 1.4961x; 1.0894x over previous
import functools

import jax
import jax.numpy as jnp
from jax import lax
from jax.experimental import pallas as pl
from jax.experimental.pallas import tpu as pltpu
from jax.experimental.pallas import tpu_sc as plsc

NUM_EMB = 8192
DIM = 256
BM = 512


_PARTS = ((0, 2736), (2736, 5472), (5472, NUM_EMB))


def _argmin_body(x_ref, e_ref, idx_ref, loss_ref, e2a_ref, e2b_ref, e2c_ref):
    i = pl.program_id(0)
    e2_refs = (e2a_ref, e2b_ref, e2c_ref)

    @pl.when(i == 0)
    def _init():
        for (lo, hi), r in zip(_PARTS, e2_refs):
            e_p = e_ref[lo:hi, :]
            r[...] = jnp.sum(e_p * e_p, axis=1, keepdims=True).reshape(1, hi - lo)
        loss_ref[...] = jnp.zeros((1, 1), jnp.float32)

    x = x_ref[...]
    x2 = jnp.sum(x * x, axis=1, keepdims=True)
    accv = acci = exact = None
    for (lo, hi), e2_r in zip(_PARTS, e2_refs):
        e_p = e_ref[lo:hi, :]
        m = lax.dot_general(x, e_p, (((1,), (1,)), ((), ())),
                            preferred_element_type=jnp.float32)
        d = (x2 + e2_r[...]) - 2.0 * m
        mp = jnp.min(d, axis=1, keepdims=True)
        col = lax.broadcasted_iota(jnp.int32, d.shape, 1) + lo
        ip = jnp.min(jnp.where(d == mp, col, NUM_EMB), axis=1)
        mp = mp[:, 0]
        mpr = mp.astype(jnp.bfloat16).astype(jnp.float32)
        if accv is None:
            accv, acci, exact = mpr, ip, mp
        else:
            take = mp < accv
            accv = jnp.where(take, mpr, accv)
            acci = jnp.where(take, ip, acci)
            exact = jnp.where(take, mp, exact)
    idx_ref[0, 0, :] = acci
    loss_ref[...] += jnp.sum(exact).reshape(1, 1)


def _vq_argmin_call(flat, embedding):
    n = flat.shape[0]
    nblk = n // BM
    grid = (nblk,)
    idx_out = jax.ShapeDtypeStruct((nblk, 1, BM), jnp.int32)
    loss_out = jax.ShapeDtypeStruct((1, 1), jnp.float32)
    idx3, loss = pl.pallas_call(
        _argmin_body,
        grid=grid,
        in_specs=[
            pl.BlockSpec((BM, DIM), lambda i: (i, 0)),
            pl.BlockSpec((NUM_EMB, DIM), lambda i: (0, 0)),
        ],
        out_specs=[
            pl.BlockSpec((1, 1, BM), lambda i: (i, 0, 0)),
            pl.BlockSpec((1, 1), lambda i: (0, 0)),
        ],
        out_shape=[idx_out, loss_out],
        scratch_shapes=[pltpu.VMEM((1, hi - lo), jnp.float32)
                        for lo, hi in _PARTS],
    )(flat, embedding)
    return idx3.reshape(n), loss[0, 0]


def _make_gather(nrows):
    info = plsc.get_sparse_core_info()
    nc, ns = info.num_cores, info.num_subcores
    nw = nc * ns
    b_per_w = nrows // nw
    ch = 128
    n_ch = b_per_w // ch
    mesh = plsc.VectorSubcoreMesh(core_axis_name="c", subcore_axis_name="s")

    @functools.partial(
        pl.kernel,
        mesh=mesh,
        out_type=jax.ShapeDtypeStruct((nrows, DIM), jnp.float32),
        scratch_types=[
            pltpu.VMEM((ch,), jnp.int32),
            pltpu.VMEM((ch, DIM), jnp.float32),
            pltpu.SemaphoreType.DMA,
        ],
    )
    def gather_k(table_hbm, idx_hbm, out_hbm, idx_v, rows_v, sem):
        wid = lax.axis_index("s") * nc + lax.axis_index("c")
        base = wid * b_per_w
        for t in range(n_ch):
            o = base + t * ch
            pltpu.sync_copy(idx_hbm.at[pl.ds(o, ch)], idx_v)
            pltpu.async_copy(table_hbm.at[idx_v], rows_v, sem).wait()
            pltpu.sync_copy(rows_v, out_hbm.at[pl.ds(o, ch)])

    return gather_k


def kernel(inputs, embedding):
    flat = inputs.reshape(-1, DIM)
    n = flat.shape[0]
    idx_flat, loss_sum = _vq_argmin_call(flat, embedding)
    quantized_flat = _make_gather(n)(embedding, idx_flat)
    quantized = quantized_flat.reshape(inputs.shape)
    loss = loss_sum * (1.25 / (n * DIM))
    return (quantized, idx_flat[:, None], loss)


# speedup vs baseline: 1.5559x; 1.0400x over previous
import functools

import jax
import jax.numpy as jnp
from jax import lax
from jax.experimental import pallas as pl
from jax.experimental.pallas import tpu as pltpu
from jax.experimental.pallas import tpu_sc as plsc

NUM_EMB = 8192
DIM = 256
BM = 1024


_PARTS = ((0, 2736), (2736, 5472), (5472, NUM_EMB))


def _argmin_body(x_ref, e_ref, idx_ref, loss_ref, e2a_ref, e2b_ref, e2c_ref):
    i = pl.program_id(0)
    e2_refs = (e2a_ref, e2b_ref, e2c_ref)

    @pl.when(i == 0)
    def _init():
        for (lo, hi), r in zip(_PARTS, e2_refs):
            e_p = e_ref[lo:hi, :]
            r[...] = jnp.sum(e_p * e_p, axis=1, keepdims=True).reshape(1, hi - lo)
        loss_ref[...] = jnp.zeros((1, 1), jnp.float32)

    x = x_ref[...]
    x2 = jnp.sum(x * x, axis=1, keepdims=True)
    accv = acci = exact = None
    for (lo, hi), e2_r in zip(_PARTS, e2_refs):
        e_p = e_ref[lo:hi, :]
        m = lax.dot_general(x, e_p, (((1,), (1,)), ((), ())),
                            preferred_element_type=jnp.float32)
        d = (x2 + e2_r[...]) - 2.0 * m
        mp = jnp.min(d, axis=1, keepdims=True)
        col = lax.broadcasted_iota(jnp.int32, d.shape, 1) + lo
        ip = jnp.min(jnp.where(d == mp, col, NUM_EMB), axis=1)
        mp = mp[:, 0]
        mpr = mp.astype(jnp.bfloat16).astype(jnp.float32)
        if accv is None:
            accv, acci, exact = mpr, ip, mp
        else:
            take = mp < accv
            accv = jnp.where(take, mpr, accv)
            acci = jnp.where(take, ip, acci)
            exact = jnp.where(take, mp, exact)
    idx_ref[0, 0, :] = acci
    loss_ref[...] += jnp.sum(exact).reshape(1, 1)


def _vq_argmin_call(flat, embedding):
    n = flat.shape[0]
    nblk = n // BM
    grid = (nblk,)
    idx_out = jax.ShapeDtypeStruct((nblk, 1, BM), jnp.int32)
    loss_out = jax.ShapeDtypeStruct((1, 1), jnp.float32)
    idx3, loss = pl.pallas_call(
        _argmin_body,
        grid=grid,
        in_specs=[
            pl.BlockSpec((BM, DIM), lambda i: (i, 0)),
            pl.BlockSpec((NUM_EMB, DIM), lambda i: (0, 0)),
        ],
        out_specs=[
            pl.BlockSpec((1, 1, BM), lambda i: (i, 0, 0)),
            pl.BlockSpec((1, 1), lambda i: (0, 0)),
        ],
        out_shape=[idx_out, loss_out],
        scratch_shapes=[pltpu.VMEM((1, hi - lo), jnp.float32)
                        for lo, hi in _PARTS],
    )(flat, embedding)
    return idx3.reshape(n), loss[0, 0]


def _make_gather(nrows):
    info = plsc.get_sparse_core_info()
    nc, ns = info.num_cores, info.num_subcores
    nw = nc * ns
    b_per_w = nrows // nw
    ch = 128
    n_ch = b_per_w // ch
    mesh = plsc.VectorSubcoreMesh(core_axis_name="c", subcore_axis_name="s")

    @functools.partial(
        pl.kernel,
        mesh=mesh,
        out_type=jax.ShapeDtypeStruct((nrows, DIM), jnp.float32),
        scratch_types=[
            pltpu.VMEM((ch,), jnp.int32),
            pltpu.VMEM((ch, DIM), jnp.float32),
            pltpu.SemaphoreType.DMA,
        ],
    )
    def gather_k(table_hbm, idx_hbm, out_hbm, idx_v, rows_v, sem):
        wid = lax.axis_index("s") * nc + lax.axis_index("c")
        base = wid * b_per_w
        for t in range(n_ch):
            o = base + t * ch
            pltpu.sync_copy(idx_hbm.at[pl.ds(o, ch)], idx_v)
            pltpu.async_copy(table_hbm.at[idx_v], rows_v, sem).wait()
            pltpu.sync_copy(rows_v, out_hbm.at[pl.ds(o, ch)])

    return gather_k


def kernel(inputs, embedding):
    flat = inputs.reshape(-1, DIM)
    n = flat.shape[0]
    idx_flat, loss_sum = _vq_argmin_call(flat, embedding)
    quantized_flat = _make_gather(n)(embedding, idx_flat)
    quantized = quantized_flat.reshape(inputs.shape)
    loss = loss_sum * (1.25 / (n * DIM))
    return (quantized, idx_flat[:, None], loss)


# speedup vs baseline: 1.6538x; 1.0629x over previous
import functools

import jax
import jax.numpy as jnp
from jax import lax
from jax.experimental import pallas as pl
from jax.experimental.pallas import tpu as pltpu
from jax.experimental.pallas import tpu_sc as plsc

NUM_EMB = 8192
DIM = 256
BM = 2048


_PARTS = ((0, 2736), (2736, 5472), (5472, NUM_EMB))


def _argmin_body(x_ref, e_ref, idx_ref, loss_ref, e2a_ref, e2b_ref, e2c_ref):
    i = pl.program_id(0)
    e2_refs = (e2a_ref, e2b_ref, e2c_ref)

    @pl.when(i == 0)
    def _init():
        for (lo, hi), r in zip(_PARTS, e2_refs):
            e_p = e_ref[lo:hi, :]
            r[...] = jnp.sum(e_p * e_p, axis=1, keepdims=True).reshape(1, hi - lo)
        loss_ref[...] = jnp.zeros((1, 1), jnp.float32)

    x = x_ref[...]
    x2 = jnp.sum(x * x, axis=1, keepdims=True)
    accv = acci = exact = None
    for (lo, hi), e2_r in zip(_PARTS, e2_refs):
        e_p = e_ref[lo:hi, :]
        m = lax.dot_general(x, e_p, (((1,), (1,)), ((), ())),
                            preferred_element_type=jnp.float32)
        d = (x2 + e2_r[...]) - 2.0 * m
        mp = jnp.min(d, axis=1, keepdims=True)
        col = lax.broadcasted_iota(jnp.int32, d.shape, 1) + lo
        ip = jnp.min(jnp.where(d == mp, col, NUM_EMB), axis=1)
        mp = mp[:, 0]
        mpr = mp.astype(jnp.bfloat16).astype(jnp.float32)
        if accv is None:
            accv, acci, exact = mpr, ip, mp
        else:
            take = mp < accv
            accv = jnp.where(take, mpr, accv)
            acci = jnp.where(take, ip, acci)
            exact = jnp.where(take, mp, exact)
    idx_ref[0, 0, :] = acci
    loss_ref[...] += jnp.sum(exact).reshape(1, 1)


def _vq_argmin_call(flat, embedding):
    n = flat.shape[0]
    nblk = n // BM
    grid = (nblk,)
    idx_out = jax.ShapeDtypeStruct((nblk, 1, BM), jnp.int32)
    loss_out = jax.ShapeDtypeStruct((1, 1), jnp.float32)
    idx3, loss = pl.pallas_call(
        _argmin_body,
        grid=grid,
        in_specs=[
            pl.BlockSpec((BM, DIM), lambda i: (i, 0)),
            pl.BlockSpec((NUM_EMB, DIM), lambda i: (0, 0)),
        ],
        out_specs=[
            pl.BlockSpec((1, 1, BM), lambda i: (i, 0, 0)),
            pl.BlockSpec((1, 1), lambda i: (0, 0)),
        ],
        out_shape=[idx_out, loss_out],
        scratch_shapes=[pltpu.VMEM((1, hi - lo), jnp.float32)
                        for lo, hi in _PARTS],
    )(flat, embedding)
    return idx3.reshape(n), loss[0, 0]


def _make_gather(nrows):
    info = plsc.get_sparse_core_info()
    nc, ns = info.num_cores, info.num_subcores
    nw = nc * ns
    b_per_w = nrows // nw
    ch = 128
    n_ch = b_per_w // ch
    mesh = plsc.VectorSubcoreMesh(core_axis_name="c", subcore_axis_name="s")

    @functools.partial(
        pl.kernel,
        mesh=mesh,
        out_type=jax.ShapeDtypeStruct((nrows, DIM), jnp.float32),
        scratch_types=[
            pltpu.VMEM((ch,), jnp.int32),
            pltpu.VMEM((ch, DIM), jnp.float32),
            pltpu.SemaphoreType.DMA,
        ],
    )
    def gather_k(table_hbm, idx_hbm, out_hbm, idx_v, rows_v, sem):
        wid = lax.axis_index("s") * nc + lax.axis_index("c")
        base = wid * b_per_w
        for t in range(n_ch):
            o = base + t * ch
            pltpu.sync_copy(idx_hbm.at[pl.ds(o, ch)], idx_v)
            pltpu.async_copy(table_hbm.at[idx_v], rows_v, sem).wait()
            pltpu.sync_copy(rows_v, out_hbm.at[pl.ds(o, ch)])

    return gather_k


def kernel(inputs, embedding):
    flat = inputs.reshape(-1, DIM)
    n = flat.shape[0]
    idx_flat, loss_sum = _vq_argmin_call(flat, embedding)
    quantized_flat = _make_gather(n)(embedding, idx_flat)
    quantized = quantized_flat.reshape(inputs.shape)
    loss = loss_sum * (1.25 / (n * DIM))
    return (quantized, idx_flat[:, None], loss)
